# baseline (speedup 1.0000x reference)
import jax
import jax.numpy as jnp
from jax import lax
from jax.experimental import pallas as pl
from jax.experimental.pallas import tpu as pltpu
from jax.experimental.pallas import tpu_sc as plsc

_N = 10000
_D = 128
_C = 40
_NP = 10112
_PADROW = 10008
_E = 320000
_NW = 32
_K = 128
_CH = 80
_CHI = 82
_RPT = _NP // 16
_RB = 1024


def _sc_agg_body(table, pk_r, zrows, part,
                 pk_v, src_ring, dst_ring, rows_v, acc_sh, g0, g1, s0, s1):
    gsem = (g0, g1)
    ssem = (s0, s1)
    c = lax.axis_index("c")
    s = lax.axis_index("s")
    wid = c * 16 + s
    base = s * _RPT
    pltpu.sync_copy(zrows.at[pl.ds(base, _RPT)], acc_sh.at[pl.ds(base, _RPT)])
    pltpu.sync_copy(pk_r.at[wid], pk_v)
    plsc.subcore_barrier()

    def unpack(j, b):
        for i in range(8):
            v = pk_v[j, pl.ds(16 * i, 16)]
            src_ring[b * 8, pl.ds(16 * i, 16)] = lax.shift_right_logical(v, 14)
            dst_ring[b * 8, pl.ds(16 * i, 16)] = v & 0x3FFF

    def start_gather(b):
        pltpu.async_copy(table.at[src_ring.at[b * 8]], rows_v.at[b], gsem[b])

    def wait_gather(b):
        pltpu.make_async_copy(table.at[src_ring.at[0]], rows_v.at[b],
                              gsem[b]).wait()

    def start_scatter(b):
        pltpu.async_copy(rows_v.at[b], acc_sh.at[dst_ring.at[b * 8]], ssem[b],
                         add=True)

    def wait_scatter(b):
        pltpu.make_async_copy(rows_v.at[b], acc_sh.at[dst_ring.at[0]],
                              ssem[b]).wait()

    unpack(0, 0)
    unpack(1, 1)
    start_gather(0)
    start_gather(1)

    def pair(t, carry):
        j0 = 2 * t
        wait_gather(0)
        start_scatter(0)
        wait_scatter(0)
        unpack(j0 + 2, 0)
        start_gather(0)
        wait_gather(1)
        start_scatter(1)
        wait_scatter(1)
        unpack(j0 + 3, 1)
        start_gather(1)
        return carry

    lax.fori_loop(0, _CH // 2, pair, 0)
    wait_gather(0)
    wait_gather(1)
    plsc.subcore_barrier()
    pltpu.sync_copy(acc_sh.at[pl.ds(base, _RPT)], part.at[c, pl.ds(base, _RPT)])


def _sc_deg_body(dst_r, zrows, ones_hbm, degpart, dst_v, ones_v, deg_sh, sem):
    c = lax.axis_index("c")
    s = lax.axis_index("s")
    wid = c * 16 + s
    base = s * _RPT
    pltpu.sync_copy(zrows.at[pl.ds(base, _RPT)], deg_sh.at[pl.ds(base, _RPT)])
    pltpu.sync_copy(ones_hbm, ones_v)
    pltpu.sync_copy(dst_r.at[wid], dst_v)
    plsc.subcore_barrier()

    def start_scatter(j):
        pltpu.async_copy(ones_v, deg_sh.at[dst_v.at[j]], sem, add=True)

    def wait_one():
        pltpu.make_async_copy(ones_v, deg_sh.at[dst_v.at[0]], sem).wait()

    for b in range(4):
        start_scatter(b)

    def step(j, carry):
        wait_one()
        start_scatter(j + 4)
        return carry

    lax.fori_loop(0, _CH - 4, step, 0)
    for b in range(4):
        wait_one()
    plsc.subcore_barrier()
    pltpu.sync_copy(deg_sh.at[pl.ds(base, _RPT)], degpart.at[c, pl.ds(base, _RPT)])


_sc_mesh = plsc.VectorSubcoreMesh(core_axis_name="c", subcore_axis_name="s")

_sc_agg = pl.kernel(
    _sc_agg_body,
    out_type=jax.ShapeDtypeStruct((2, _NP, _D), jnp.float32),
    mesh=_sc_mesh,
    scratch_types=[
        pltpu.VMEM((_CHI, _K), jnp.int32),
        pltpu.VMEM((16, _K), jnp.int32),
        pltpu.VMEM((16, _K), jnp.int32),
        pltpu.VMEM((2, _K, _D), jnp.float32),
        pltpu.VMEM_SHARED((_NP, _D), jnp.float32),
    ] + [pltpu.SemaphoreType.DMA] * 4,
)

_sc_deg = pl.kernel(
    _sc_deg_body,
    out_type=jax.ShapeDtypeStruct((2, _NP, _D), jnp.float32),
    mesh=_sc_mesh,
    scratch_types=[
        pltpu.VMEM((_CH, _K), jnp.int32),
        pltpu.VMEM((_K, _D), jnp.float32),
        pltpu.VMEM_SHARED((_NP, _D), jnp.float32),
        pltpu.SemaphoreType.DMA,
    ],
)


def _mm(a, b):
    return jnp.dot(a, b, preferred_element_type=jnp.float32)


def _dense1_body(part, degpart, xa, wl, wr, b, out):
    p = part[...]
    agg = p[0] + p[1]
    d = degpart[...]
    deg = d[0, :, 0:1] + d[1, :, 0:1]
    mean = agg / jnp.maximum(deg, 1.0)
    h = _mm(mean, wl[...]) + _mm(xa[...], wr[...]) + b[...]
    out[...] = jnp.maximum(h, 0.0)


_dense1 = pl.pallas_call(
    _dense1_body,
    grid=(10,),
    in_specs=[
        pl.BlockSpec((2, _RB, _D), lambda i: (0, i, 0)),
        pl.BlockSpec((2, _RB, _D), lambda i: (0, i, 0)),
        pl.BlockSpec((_RB, _D), lambda i: (i, 0)),
        pl.BlockSpec((_D, _D), lambda i: (0, 0)),
        pl.BlockSpec((_D, _D), lambda i: (0, 0)),
        pl.BlockSpec((1, _D), lambda i: (0, 0)),
    ],
    out_specs=pl.BlockSpec((_RB, _D), lambda i: (i, 0)),
    out_shape=jax.ShapeDtypeStruct((_NP, _D), jnp.float32),
)


def _dense2_body(part, degpart, h1a, w2l, w2r, b2, wc1, bc1, wc2, bc2, out):
    p = part[...]
    agg = p[0] + p[1]
    d = degpart[...]
    deg = d[0, :, 0:1] + d[1, :, 0:1]
    mean = agg / jnp.maximum(deg, 1.0)
    h2 = _mm(mean, w2l[...]) + _mm(h1a[...], w2r[...]) + b2[...]
    h2 = jnp.maximum(h2, 0.0)
    z = _mm(h2, wc1[...]) + bc1[...]
    z = _mm(z, wc2[...]) + bc2[...]
    m = jnp.max(z, axis=1, keepdims=True)
    ez = jnp.exp(z - m)
    out[...] = z - m - jnp.log(jnp.sum(ez, axis=1, keepdims=True))


_dense2 = pl.pallas_call(
    _dense2_body,
    grid=(10,),
    in_specs=[
        pl.BlockSpec((2, _RB, _D), lambda i: (0, i, 0)),
        pl.BlockSpec((2, _RB, _D), lambda i: (0, i, 0)),
        pl.BlockSpec((_RB, _D), lambda i: (i, 0)),
        pl.BlockSpec((_D, _D), lambda i: (0, 0)),
        pl.BlockSpec((_D, _D), lambda i: (0, 0)),
        pl.BlockSpec((1, _D), lambda i: (0, 0)),
        pl.BlockSpec((_D, _D), lambda i: (0, 0)),
        pl.BlockSpec((1, _D), lambda i: (0, 0)),
        pl.BlockSpec((_D, _C), lambda i: (0, 0)),
        pl.BlockSpec((1, _C), lambda i: (0, 0)),
    ],
    out_specs=pl.BlockSpec((_RB, _C), lambda i: (i, 0)),
    out_shape=jax.ShapeDtypeStruct((_N, _C), jnp.float32),
)


def kernel(x, edge_index, W1l, b1l, W1r, W2l, b2l, W2r, Wc1, bc1, Wc2, bc2):
    f32 = jnp.float32
    i32 = jnp.int32
    src = edge_index[0]
    dst = edge_index[1]
    pad = _NW * _CH * _K - _E
    pk = jnp.left_shift(src.astype(i32), 14) | dst.astype(i32)
    pk_r = jnp.concatenate([
        jnp.concatenate([pk, jnp.full((pad,), _PADROW, i32)]).reshape(_NW, _CH, _K),
        jnp.full((_NW, _CHI - _CH, _K), _PADROW, i32)], axis=1)
    dst_r = jnp.concatenate([dst, jnp.full((pad,), _PADROW, i32)]).reshape(_NW, _CH, _K)
    xa = jnp.zeros((_NP, _D), f32).at[:_N].set(x)
    zrows = jnp.zeros((_NP, _D), f32)
    ones = jnp.ones((_K, _D), f32)

    degpart = _sc_deg(dst_r, zrows, ones)
    part1 = _sc_agg(xa, pk_r, zrows)
    h1a = _dense1(part1, degpart, xa, W1l.T, W1r.T, b1l.reshape(1, -1))
    part2 = _sc_agg(h1a, pk_r, zrows)
    out = _dense2(part2, degpart, h1a, W2l.T, W2r.T, b2l.reshape(1, -1),
                  Wc1.T, bc1.reshape(1, -1), Wc2.T, bc2.reshape(1, -1))
    return out
